# Initial kernel scaffold; baseline (speedup 1.0000x reference)
import jax
import jax.numpy as jnp
from jax.experimental import pallas as pl


def kernel(features, edge_index, W0, b0, W1, b1, W2, b2):
    raise NotImplementedError("write your pallas kernel here")


# speedup vs baseline: 4.5238x; 4.5238x over previous
import functools

import jax
import jax.numpy as jnp
from jax import lax
from jax.experimental import pallas as pl
from jax.experimental.pallas import tpu as pltpu
from jax.experimental.pallas import tpu_sc as plsc

N = 10000
E = 320000
D = 128
NC = 2
NS = 16
NW = NC * NS
EPW = E // NW
B = 80
NB = EPW // B
ACCR = 10240
ZPT = ACCR // NS
ZR = 128
WPT = 624


def _agg_body(h_hbm, src_hbm, dst_hbm, out_hbm, acc_sh, zbuf, srcv, dstv, rows, sem):
    c = lax.axis_index("c")
    s = lax.axis_index("s")
    w = s * NC + c

    def zfill(i, carry):
        zbuf[i // 8, pl.ds((i % 8) * 16, 16)] = jnp.zeros((16,), jnp.float32)
        return carry

    lax.fori_loop(0, ZR * 8, zfill, 0)
    for j in range(ZPT // ZR):
        pltpu.sync_copy(zbuf, acc_sh.at[pl.ds(s * ZPT + j * ZR, ZR)])
    plsc.subcore_barrier()

    base = w * EPW

    def body(i, carry):
        off = base + i * B
        pltpu.sync_copy(src_hbm.at[pl.ds(off, B)], srcv)
        pltpu.sync_copy(dst_hbm.at[pl.ds(off, B)], dstv)
        pltpu.async_copy(h_hbm.at[srcv], rows, sem).wait()
        pltpu.sync_copy(rows, acc_sh.at[dstv], add=True)
        return carry

    lax.fori_loop(0, NB, body, 0)
    plsc.subcore_barrier()
    pltpu.sync_copy(acc_sh.at[pl.ds(s * WPT, WPT)],
                    out_hbm.at[c, pl.ds(s * WPT, WPT)])

    @pl.when(s == 0)
    def _tail():
        pltpu.sync_copy(acc_sh.at[pl.ds(NS * WPT, N - NS * WPT)],
                        out_hbm.at[c, pl.ds(NS * WPT, N - NS * WPT)])


_agg = pl.kernel(
    _agg_body,
    mesh=plsc.VectorSubcoreMesh(core_axis_name="c", subcore_axis_name="s"),
    out_type=jax.ShapeDtypeStruct((NC, N, D), jnp.float32),
    scratch_types=[
        pltpu.VMEM_SHARED((ACCR, D), jnp.float32),
        pltpu.VMEM((ZR, D), jnp.float32),
        pltpu.VMEM((B,), jnp.int32),
        pltpu.VMEM((B,), jnp.int32),
        pltpu.VMEM((B, D), jnp.float32),
        pltpu.SemaphoreType.DMA,
    ],
)


_GB = 1000


def _mm_body(x_ref, w_ref, o_ref):
    o_ref[...] = jnp.dot(x_ref[...], w_ref[...],
                         preferred_element_type=jnp.float32)


def _relu_mm_body(p_ref, b_ref, w_ref, o_ref):
    h = jnp.maximum(p_ref[0] + p_ref[1] + b_ref[...], 0.0)
    o_ref[...] = jnp.dot(h, w_ref[...], preferred_element_type=jnp.float32)


def _relu_add_body(p_ref, b_ref, o_ref):
    o_ref[...] = jnp.maximum(p_ref[0] + p_ref[1] + b_ref[...], 0.0)


def _add_mm_body(p_ref, b_ref, w_ref, o_ref):
    o_ref[...] = jnp.dot(p_ref[0] + p_ref[1], w_ref[...],
                         preferred_element_type=jnp.float32) + b_ref[...]


def _mm(x, w):
    return pl.pallas_call(
        _mm_body,
        grid=(N // _GB,),
        in_specs=[pl.BlockSpec((_GB, D), lambda i: (i, 0)),
                  pl.BlockSpec((D, D), lambda i: (0, 0))],
        out_specs=pl.BlockSpec((_GB, D), lambda i: (i, 0)),
        out_shape=jax.ShapeDtypeStruct((N, D), jnp.float32),
    )(x, w)


def _relu_mm(p, b, w):
    return pl.pallas_call(
        _relu_mm_body,
        grid=(N // _GB,),
        in_specs=[pl.BlockSpec((2, _GB, D), lambda i: (0, i, 0)),
                  pl.BlockSpec((1, D), lambda i: (0, 0)),
                  pl.BlockSpec((D, D), lambda i: (0, 0))],
        out_specs=pl.BlockSpec((_GB, D), lambda i: (i, 0)),
        out_shape=jax.ShapeDtypeStruct((N, D), jnp.float32),
    )(p, b, w)


def _relu_add(p, b):
    return pl.pallas_call(
        _relu_add_body,
        grid=(N // _GB,),
        in_specs=[pl.BlockSpec((2, _GB, D), lambda i: (0, i, 0)),
                  pl.BlockSpec((1, D), lambda i: (0, 0))],
        out_specs=pl.BlockSpec((_GB, D), lambda i: (i, 0)),
        out_shape=jax.ShapeDtypeStruct((N, D), jnp.float32),
    )(p, b)


def _add_mm(p, b, w):
    return pl.pallas_call(
        _add_mm_body,
        grid=(N // _GB,),
        in_specs=[pl.BlockSpec((2, _GB, D), lambda i: (0, i, 0)),
                  pl.BlockSpec((1, D), lambda i: (0, 0)),
                  pl.BlockSpec((D, D), lambda i: (0, 0))],
        out_specs=pl.BlockSpec((_GB, D), lambda i: (i, 0)),
        out_shape=jax.ShapeDtypeStruct((N, D), jnp.float32),
    )(p, b, w)


def kernel(features, edge_index, W0, b0, W1, b1, W2, b2):
    src = edge_index[0]
    dst = edge_index[1]
    w2p = jnp.zeros((D, D), jnp.float32).at[:, : W2.shape[1]].set(W2)
    b2p = jnp.zeros((1, D), jnp.float32).at[:, : W2.shape[1]].set(b2)

    y0 = _mm(features, W0)
    p0 = _agg(y0, src, dst)
    y1 = _relu_mm(p0, b0.reshape(1, D), W1)
    p1 = _agg(y1, src, dst)
    h1 = _relu_add(p1, b1.reshape(1, D))
    p2 = _agg(h1, src, dst)
    out = _add_mm(p2, b2p, w2p)
    return out[:, : W2.shape[1]]
